# baseline (speedup 1.0000x reference)
import functools

import jax
import jax.numpy as jnp
from jax import lax
from jax.experimental import pallas as pl
from jax.experimental.pallas import tpu as pltpu
from jax.experimental.pallas import tpu_sc as plsc

VOCAB = 1000000
HIDDEN = 64
BATCH = 4096
SEQ = 200
SEQP = 208
NC = 2
NS = 16
NW = NC * NS

NPAIR = VOCAB // 2
NBLK = VOCAB // 128
REM = VOCAB - NBLK * 128
B_PER_W = BATCH // NW


def _transpose_block(src, dst, npair):
  def one_row(q):
    for k in range(8):
      h = jax.lax.iota(jnp.int32, 16) + (k % 4) * 16
      tau = jnp.full((16,), 0, jnp.int32) + (2 * q + k // 4)
      v = plsc.load_gather(src, [h, tau])
      dst[q, pl.ds(k * 16, 16)] = v
  pl.loop(0, npair)(one_row)


def _transpose_kernel(embt_hbm, embr_hbm, tab_hbm, src0, src1, dst0, dst1,
                      rem_v, isem0, isem1, osem0, osem1):
  wid = lax.axis_index("s") * NC + lax.axis_index("c")
  srcs = (src0, src1)
  dsts = (dst0, dst1)
  isems = (isem0, isem1)
  osems = (osem0, osem1)
  nblocks = (NBLK - 1 - wid) // NW + 1

  def in_copy(b, i):
    tc = (2 * i + b) * NW + wid
    return pltpu.make_async_copy(
        embt_hbm.at[:, pl.ds(pl.multiple_of(tc * 128, 128), 128)],
        srcs[b], isems[b])

  def out_copy(b, i):
    tc = (2 * i + b) * NW + wid
    return pltpu.make_async_copy(
        dsts[b], tab_hbm.at[pl.ds(pl.multiple_of(tc * 64, 64), 64)],
        osems[b])

  def step(b, i):
    in_copy(b, i).wait()
    _transpose_block(srcs[b], dsts[b], 64)
    @pl.when((2 * i + b + 2) * NW + wid < NBLK)
    def _():
      in_copy(b, i + 1).start()
    @pl.when(i >= 1)
    def _():
      out_copy(b, i - 1).wait()
    out_copy(b, i).start()

  @pl.when(wid < NBLK)
  def _():
    in_copy(0, 0).start()
  @pl.when(NW + wid < NBLK)
  def _():
    in_copy(1, 0).start()

  def body(i):
    @pl.when(2 * i * NW + wid < NBLK)
    def _():
      step(0, i)
    @pl.when((2 * i + 1) * NW + wid < NBLK)
    def _():
      step(1, i)

  pl.loop(0, (NBLK // NW) // 2 + 1)(body)

  lastpair = (nblocks - 1) // 2
  @pl.when(nblocks >= 1)
  def _():
    @pl.when(jax.lax.rem(nblocks - 1, 2) == 0)
    def _():
      out_copy(0, lastpair).wait()
      @pl.when(nblocks >= 2)
      def _():
        out_copy(1, lastpair - 1).wait()
    @pl.when(jax.lax.rem(nblocks - 1, 2) == 1)
    def _():
      out_copy(1, lastpair).wait()
      out_copy(0, lastpair).wait()

  @pl.when(wid == 0)
  def _():
    pltpu.sync_copy(embr_hbm, rem_v)
    _transpose_block(rem_v, dst0, REM // 2)
    pltpu.sync_copy(dst0.at[pl.ds(0, REM // 2)],
                    tab_hbm.at[pl.ds(NBLK * 64, REM // 2)])


def _gather_kernel(tab_hbm, xp_hbm, out_hbm, idx_v, pidx_v, hoff_v, pair0,
                   pair1, outb0, outb1, gsem0, gsem1, osem0, osem1):
  wid = lax.axis_index("s") * NC + lax.axis_index("c")
  b0 = wid * B_PER_W

  pltpu.sync_copy(xp_hbm.at[pl.ds(pl.multiple_of(b0, B_PER_W), B_PER_W)],
                  idx_v)

  pairs = (pair0, pair1)
  outbs = (outb0, outb1)
  gsems = (gsem0, gsem1)
  osems = (osem0, osem1)

  def tok_col(s, j):
    blv = jax.lax.iota(jnp.int32, 16) + j * 16
    sv = jnp.full((16,), 0, jnp.int32) + s
    return blv, plsc.load_gather(idx_v, [blv, sv])

  def mk_pidx(s, b):
    for j in range(B_PER_W // 16):
      _, tok = tok_col(s, j)
      pidx_v[b, pl.ds(j * 16, 16)] = jax.lax.shift_right_logical(tok, 1)
      hoff_v[b, pl.ds(j * 16, 16)] = jax.lax.mul(jax.lax.rem(tok, 2), 64)

  def gather(s, b):
    return pltpu.make_async_copy(
        tab_hbm.at[pidx_v.at[b]], pairs[b], gsems[b])

  def build(s, b):
    pair, outb = pairs[b], outbs[b]
    def one_h(h):
      hv = jnp.full((16,), 0, jnp.int32) + h
      for j in range(B_PER_W // 16):
        blv = jax.lax.iota(jnp.int32, 16) + j * 16
        hoff = hoff_v[b, pl.ds(j * 16, 16)]
        v = plsc.load_gather(pair, [blv, hoff + hv])
        outb[h, pl.ds(j * 16, 16)] = v
    pl.loop(0, HIDDEN)(one_h)

  def out_dma(s, b):
    return pltpu.make_async_copy(
        outbs[b], out_hbm.at[s, :, pl.ds(pl.multiple_of(b0, B_PER_W),
                                         B_PER_W)], osems[b])

  def do_s(s, b, first, last):
    if not last:
      mk_pidx(s + 1, 1 - b)
      gather(s + 1, 1 - b).start()
    gather(s, b).wait()
    if not first:
      out_dma(s - 2, b).wait()
    build(s, b)
    out_dma(s, b).start()

  mk_pidx(0, 0)
  gather(0, 0).start()
  do_s(0, 0, True, False)
  do_s(1, 1, True, False)

  def body(i):
    @pl.when(i < SEQ // 2 - 1)
    def _():
      do_s(2 * i, 0, False, False)
      do_s(2 * i + 1, 1, False, False)
    @pl.when(i == SEQ // 2 - 1)
    def _():
      do_s(2 * i, 0, False, False)
      do_s(2 * i + 1, 1, False, True)

  pl.loop(1, SEQ // 2)(body)
  out_dma(SEQ - 2, 0).wait()
  out_dma(SEQ - 1, 1).wait()


@jax.jit
def kernel(x, emb):
  mesh = plsc.VectorSubcoreMesh(core_axis_name="c", subcore_axis_name="s")
  tparams = pltpu.CompilerParams(use_tc_tiling_on_sc=True,
                                 needs_layout_passes=False)

  transpose = functools.partial(
      pl.kernel,
      mesh=mesh,
      out_type=jax.ShapeDtypeStruct((NPAIR, 128), jnp.float32),
      scratch_types=[
          pltpu.VMEM((64, 128), jnp.float32),
          pltpu.VMEM((64, 128), jnp.float32),
          pltpu.VMEM((64, 128), jnp.float32),
          pltpu.VMEM((64, 128), jnp.float32),
          pltpu.VMEM((64, 64), jnp.float32),
          pltpu.SemaphoreType.DMA,
          pltpu.SemaphoreType.DMA,
          pltpu.SemaphoreType.DMA,
          pltpu.SemaphoreType.DMA,
      ],
      compiler_params=tparams,
  )(_transpose_kernel)

  gather = functools.partial(
      pl.kernel,
      mesh=mesh,
      out_type=jax.ShapeDtypeStruct((SEQ, HIDDEN, BATCH), jnp.float32),
      scratch_types=[
          pltpu.VMEM((B_PER_W, SEQP), jnp.int32),
          pltpu.VMEM((2, B_PER_W), jnp.int32),
          pltpu.VMEM((2, B_PER_W), jnp.int32),
          pltpu.VMEM((B_PER_W, 128), jnp.float32),
          pltpu.VMEM((B_PER_W, 128), jnp.float32),
          pltpu.VMEM((HIDDEN, B_PER_W), jnp.float32),
          pltpu.VMEM((HIDDEN, B_PER_W), jnp.float32),
          pltpu.SemaphoreType.DMA,
          pltpu.SemaphoreType.DMA,
          pltpu.SemaphoreType.DMA,
          pltpu.SemaphoreType.DMA,
      ],
      compiler_params=tparams,
  )(_gather_kernel)

  embt = emb.T
  tab = transpose(embt, embt[:, NBLK * 128:])
  xh = jnp.pad(x.astype(jnp.int32), ((0, 0), (0, SEQP - SEQ)))
  return gather(tab, xh).transpose(2, 0, 1)


# speedup vs baseline: 3.7194x; 3.7194x over previous
import functools

import jax
import jax.numpy as jnp
from jax import lax
from jax.experimental import pallas as pl
from jax.experimental.pallas import tpu as pltpu
from jax.experimental.pallas import tpu_sc as plsc

VOCAB = 1000000
HIDDEN = 64
BATCH = 4096
SEQ = 200

NC = 2
NS = 16
NW = NC * NS

TOTAL = BATCH * SEQ
PER_W = TOTAL // NW
IW = 128
CHUNK = 512
NSUB = CHUNK // IW
NCHUNK = PER_W // CHUNK
IDX_ROWS = PER_W // IW


def _gather_kernel(emb_hbm, x_hbm, out_hbm, idx_v, rows0, rows1, gsem0,
                   gsem1, osem0, osem1):
  wid = lax.axis_index("s") * NC + lax.axis_index("c")
  base = wid * PER_W

  pltpu.sync_copy(x_hbm.at[pl.ds(wid * IDX_ROWS, IDX_ROWS)], idx_v)

  rows = (rows0, rows1)
  gsems = (gsem0, gsem1)
  osems = (osem0, osem1)

  def out_copy(buf, c, sem):
    return pltpu.make_async_copy(
        buf, out_hbm.at[pl.ds(base + c * CHUNK, CHUNK), pl.ds(0, HIDDEN)],
        sem)

  def do_chunk(c, b, first):
    buf, gsem, osem = rows[b], gsems[b], osems[b]
    if not first:
      out_copy(buf, c - 2, osem).wait()
    cps = [
        pltpu.make_async_copy(
            emb_hbm.at[idx_v.at[c * NSUB + j]],
            buf.at[pl.ds(j * IW, IW)], gsem)
        for j in range(NSUB)
    ]
    for cp in cps:
      cp.start()
    for cp in cps:
      cp.wait()
    out_copy(buf, c, osem).start()

  do_chunk(0, 0, True)
  do_chunk(1, 1, True)

  def body(i):
    do_chunk(2 * i, 0, False)
    do_chunk(2 * i + 1, 1, False)

  pl.loop(1, NCHUNK // 2)(body)

  out_copy(rows[0], NCHUNK - 2, osems[0]).wait()
  out_copy(rows[1], NCHUNK - 1, osems[1]).wait()


@jax.jit
def kernel(x, emb):
  mesh = plsc.VectorSubcoreMesh(core_axis_name="c", subcore_axis_name="s")
  gather = functools.partial(
      pl.kernel,
      mesh=mesh,
      out_type=jax.ShapeDtypeStruct((TOTAL, 2 * HIDDEN), jnp.float32),
      scratch_types=[
          pltpu.VMEM((IDX_ROWS, IW), jnp.int32),
          pltpu.VMEM((CHUNK, HIDDEN), jnp.float32),
          pltpu.VMEM((CHUNK, HIDDEN), jnp.float32),
          pltpu.SemaphoreType.DMA,
          pltpu.SemaphoreType.DMA,
          pltpu.SemaphoreType.DMA,
          pltpu.SemaphoreType.DMA,
      ],
      compiler_params=pltpu.CompilerParams(use_tc_tiling_on_sc=False),
  )(_gather_kernel)
  xf = x.reshape(TOTAL // IW, IW).astype(jnp.int32)
  out = gather(emb, xf)
  return out.reshape(BATCH, SEQ, 2 * HIDDEN)[:, :, :HIDDEN]


# speedup vs baseline: 3.7236x; 1.0012x over previous
import functools

import jax
import jax.numpy as jnp
from jax import lax
from jax.experimental import pallas as pl
from jax.experimental.pallas import tpu as pltpu
from jax.experimental.pallas import tpu_sc as plsc

VOCAB = 1000000
HIDDEN = 64
BATCH = 4096
SEQ = 200

NC = 2
NS = 16
NW = NC * NS

TOTAL = BATCH * SEQ
PER_W = TOTAL // NW
IW = 128
CHUNK = 640
NSUB = CHUNK // IW
NCHUNK = PER_W // CHUNK
IDX_ROWS = PER_W // IW


def _gather_kernel(emb_hbm, x_hbm, out_hbm, idx_v, rows0, rows1, gsem0,
                   gsem1, osem0, osem1, xsem):
  wid = lax.axis_index("s") * NC + lax.axis_index("c")
  base = wid * PER_W

  head = 3 * NSUB
  pltpu.sync_copy(x_hbm.at[pl.ds(wid * IDX_ROWS, head)],
                  idx_v.at[pl.ds(0, head)])
  rest = pltpu.make_async_copy(
      x_hbm.at[pl.ds(wid * IDX_ROWS + head, IDX_ROWS - head)],
      idx_v.at[pl.ds(head, IDX_ROWS - head)], xsem)
  rest.start()

  rows = (rows0, rows1)
  gsems = (gsem0, gsem1)
  osems = (osem0, osem1)

  def out_copy(buf, c, sem):
    return pltpu.make_async_copy(
        buf, out_hbm.at[pl.ds(base + c * CHUNK, CHUNK), pl.ds(0, HIDDEN)],
        sem)

  def gathers(c, b):
    buf, gsem = rows[b], gsems[b]
    return [
        pltpu.make_async_copy(
            emb_hbm.at[idx_v.at[c * NSUB + j]],
            buf.at[pl.ds(j * IW, IW)], gsem)
        for j in range(NSUB)
    ]

  def do_chunk(c, b, first, last):
    nb = 1 - b
    if not last:
      if not first:
        out_copy(rows[nb], c - 1, osems[nb]).wait()
      for cp in gathers(c + 1, nb):
        cp.start()
    for cp in gathers(c, b):
      cp.wait()
    out_copy(rows[b], c, osems[b]).start()

  for cp in gathers(0, 0):
    cp.start()
  do_chunk(0, 0, True, False)
  rest.wait()
  do_chunk(1, 1, False, False)

  def body(i):
    do_chunk(2 * i, 0, False, False)
    @pl.when(i < NCHUNK // 2 - 1)
    def _():
      do_chunk(2 * i + 1, 1, False, False)
    @pl.when(i == NCHUNK // 2 - 1)
    def _():
      do_chunk(2 * i + 1, 1, False, True)

  pl.loop(1, NCHUNK // 2)(body)

  out_copy(rows[0], NCHUNK - 2, osems[0]).wait()
  out_copy(rows[1], NCHUNK - 1, osems[1]).wait()


@jax.jit
def kernel(x, emb):
  mesh = plsc.VectorSubcoreMesh(core_axis_name="c", subcore_axis_name="s")
  gather = functools.partial(
      pl.kernel,
      mesh=mesh,
      out_type=jax.ShapeDtypeStruct((TOTAL, 2 * HIDDEN), jnp.float32),
      scratch_types=[
          pltpu.VMEM((IDX_ROWS, IW), jnp.int32),
          pltpu.VMEM((CHUNK, HIDDEN), jnp.float32),
          pltpu.VMEM((CHUNK, HIDDEN), jnp.float32),
          pltpu.SemaphoreType.DMA,
          pltpu.SemaphoreType.DMA,
          pltpu.SemaphoreType.DMA,
          pltpu.SemaphoreType.DMA,
          pltpu.SemaphoreType.DMA,
      ],
      compiler_params=pltpu.CompilerParams(use_tc_tiling_on_sc=False),
  )(_gather_kernel)
  xf = x.reshape(TOTAL // IW, IW).astype(jnp.int32)
  out = gather(emb, xf)
  return out.reshape(BATCH, SEQ, 2 * HIDDEN)[:, :, :HIDDEN]
